# Initial kernel scaffold; baseline (speedup 1.0000x reference)
import jax
import jax.numpy as jnp
from jax.experimental import pallas as pl


def kernel(source_feats, target_feats):
    raise NotImplementedError("write your pallas kernel here")


# speedup vs baseline: 1.8392x; 1.8392x over previous
import functools

import jax
import jax.numpy as jnp
from jax import lax
from jax.experimental import pallas as pl
from jax.experimental.pallas import tpu as pltpu
from jax.experimental.pallas import tpu_sc as plsc

Q = 1024
D = 1024
K = 32768
KNN = 4
KB = 1024
NKB = K // KB
NEG = float("-inf")
BIGI = 2**30


def _topk_body(src_ref, tgt_ref, idx_out_ref, vals_ref, idx_ref):
    kb = pl.program_id(0)

    @pl.when(kb == 0)
    def _init():
        vals_ref[...] = jnp.full((Q, KNN), NEG, jnp.float32)
        idx_ref[...] = jnp.zeros((Q, KNN), jnp.int32)

    s = lax.dot_general(
        src_ref[...], tgt_ref[...], (((1,), (1,)), ((), ())),
        preferred_element_type=jnp.float32)
    col = lax.broadcasted_iota(jnp.int32, (Q, KB), 1) + kb * KB

    bv, bi = [], []
    x = s
    for _ in range(KNN):
        m = jnp.max(x, axis=1, keepdims=True)
        a = jnp.min(jnp.where(x == m, col, BIGI), axis=1, keepdims=True)
        bv.append(m)
        bi.append(a)
        x = jnp.where(col == a, NEG, x)

    cat_v = jnp.concatenate([vals_ref[...]] + bv, axis=1)
    cat_i = jnp.concatenate([idx_ref[...]] + bi, axis=1)
    pos = lax.broadcasted_iota(jnp.int32, (Q, 2 * KNN), 1)
    nv, ni = [], []
    for _ in range(KNN):
        m = jnp.max(cat_v, axis=1, keepdims=True)
        p = jnp.min(jnp.where(cat_v == m, pos, BIGI), axis=1, keepdims=True)
        sel = pos == p
        nv.append(m)
        ni.append(jnp.sum(jnp.where(sel, cat_i, 0), axis=1, keepdims=True))
        cat_v = jnp.where(sel, NEG, cat_v)
    vals_ref[...] = jnp.concatenate(nv, axis=1)
    idx_ref[...] = jnp.concatenate(ni, axis=1)

    @pl.when(kb == NKB - 1)
    def _out():
        idx_out_ref[...] = idx_ref[...]


def _topk_indices(src_b16, tgt_b16):
    return pl.pallas_call(
        _topk_body,
        grid=(NKB,),
        in_specs=[
            pl.BlockSpec((Q, D), lambda k: (0, 0)),
            pl.BlockSpec((KB, D), lambda k: (k, 0)),
        ],
        out_specs=pl.BlockSpec((Q, KNN), lambda k: (0, 0)),
        out_shape=jax.ShapeDtypeStruct((Q, KNN), jnp.int32),
        scratch_shapes=[
            pltpu.VMEM((Q, KNN), jnp.float32),
            pltpu.VMEM((Q, KNN), jnp.int32),
        ],
    )(src_b16, tgt_b16)


NW = 32
QPW = Q // NW
QC = 8
NCH = QPW // QC
RC = QC * KNN


def _gather_mean_body(tgt_hbm, idx_hbm, out_hbm, idx_v, rows_v, acc_v, sem):
    c = lax.axis_index("c")
    s = lax.axis_index("s")
    wid = s * 2 + c
    qbase = wid * QPW

    def chunk(ch, carry):
        rbase = (qbase + ch * QC) * KNN
        pltpu.sync_copy(idx_hbm.at[pl.ds(rbase, RC)], idx_v)
        pltpu.async_copy(tgt_hbm.at[idx_v], rows_v, sem).wait()

        def qloop(q, carry2):
            def cloop(cc, carry3):
                sl = pl.ds(cc * 16, 16)
                v = ((rows_v[q * KNN + 0, sl] + rows_v[q * KNN + 1, sl])
                     + (rows_v[q * KNN + 2, sl] + rows_v[q * KNN + 3, sl]))
                acc_v[q, sl] = v * 0.25
                return carry3
            return lax.fori_loop(0, D // 16, cloop, carry2)
        lax.fori_loop(0, QC, qloop, 0)
        pltpu.sync_copy(acc_v, out_hbm.at[pl.ds(qbase + ch * QC, QC)])
        return carry
    lax.fori_loop(0, NCH, chunk, 0)


def _gather_mean(target_feats, idx_flat):
    mesh = plsc.VectorSubcoreMesh(core_axis_name="c", subcore_axis_name="s")
    f = functools.partial(
        pl.kernel,
        mesh=mesh,
        out_type=jax.ShapeDtypeStruct((Q, D), jnp.float32),
        scratch_types=[
            pltpu.VMEM((RC,), jnp.int32),
            pltpu.VMEM((RC, D), jnp.float32),
            pltpu.VMEM((QC, D), jnp.float32),
            pltpu.SemaphoreType.DMA,
        ],
    )(_gather_mean_body)
    return f(target_feats, idx_flat)


def kernel(source_feats, target_feats):
    eps = 1e-12
    src_norm = source_feats / jnp.maximum(
        jnp.linalg.norm(source_feats, axis=1, keepdims=True), eps)
    tgt_norm = target_feats / jnp.maximum(
        jnp.linalg.norm(target_feats, axis=1, keepdims=True), eps)
    idx = _topk_indices(src_norm.astype(jnp.bfloat16),
                        tgt_norm.astype(jnp.bfloat16))
    return _gather_mean(target_feats, idx.reshape(Q * KNN))


# speedup vs baseline: 3.2877x; 1.7875x over previous
import functools

import jax
import jax.numpy as jnp
from jax import lax
from jax.experimental import pallas as pl
from jax.experimental.pallas import tpu as pltpu
from jax.experimental.pallas import tpu_sc as plsc

Q = 1024
D = 1024
K = 32768
KNN = 4
KB = 1024
NKB = K // KB
NEG = float("-inf")
BIGI = 2**30


BIGF = float(2**24)
RW = 128


def _topk_body(src_ref, tgt_ref, idx_out_ref, rv_ref, ri_ref):
    kb = pl.program_id(0)

    @pl.when(kb == 0)
    def _init():
        rv_ref[...] = jnp.full((Q, RW), NEG, jnp.float32)
        ri_ref[...] = jnp.full((Q, RW), BIGF, jnp.float32)

    t = tgt_ref[...]
    nrm = jnp.maximum(jnp.sqrt(jnp.sum(t * t, axis=1, keepdims=True)), 1e-12)
    tb = (t / nrm).astype(jnp.bfloat16)
    s = lax.dot_general(
        src_ref[...], tb, (((1,), (1,)), ((), ())),
        preferred_element_type=jnp.float32)

    colf = (lax.broadcasted_iota(jnp.int32, (Q, KB), 1).astype(jnp.float32)
            + jnp.float32(kb * KB))

    x = s
    rv = rv_ref[...]
    ri = ri_ref[...]
    ms, as_ = [], []
    for _ in range(KNN):
        m = jnp.maximum(jnp.max(x, axis=1, keepdims=True),
                        jnp.max(rv, axis=1, keepdims=True))
        a = jnp.minimum(
            jnp.min(jnp.where(x == m, colf, BIGF), axis=1, keepdims=True),
            jnp.min(jnp.where(rv == m, ri, BIGF), axis=1, keepdims=True))
        ms.append(m)
        as_.append(a)
        x = jnp.where(colf == a, NEG, x)
        rv = jnp.where(ri == a, NEG, rv)

    lane = lax.broadcasted_iota(jnp.int32, (Q, RW), 1)
    nrv = jnp.full((Q, RW), NEG, jnp.float32)
    nri = jnp.full((Q, RW), BIGF, jnp.float32)
    for t_ in range(KNN):
        nrv = jnp.where(lane == t_, ms[t_], nrv)
        nri = jnp.where(lane == t_, as_[t_], nri)
    rv_ref[...] = nrv
    ri_ref[...] = nri

    @pl.when(kb == NKB - 1)
    def _out():
        idx_out_ref[...] = nri[:, :KNN].astype(jnp.int32)


def _topk_indices(src_b16, tgt_f32):
    return pl.pallas_call(
        _topk_body,
        grid=(NKB,),
        in_specs=[
            pl.BlockSpec((Q, D), lambda k: (0, 0)),
            pl.BlockSpec((KB, D), lambda k: (k, 0)),
        ],
        out_specs=pl.BlockSpec((Q, KNN), lambda k: (0, 0)),
        out_shape=jax.ShapeDtypeStruct((Q, KNN), jnp.int32),
        scratch_shapes=[
            pltpu.VMEM((Q, RW), jnp.float32),
            pltpu.VMEM((Q, RW), jnp.float32),
        ],
    )(src_b16, tgt_f32)


NW = 32
QPW = Q // NW
QC = 8
NCH = QPW // QC
RC = QC * KNN


def _gather_mean_body(tgt_hbm, idx_hbm, out_hbm, idx_v, rows_v, acc_v, sem):
    c = lax.axis_index("c")
    s = lax.axis_index("s")
    wid = s * 2 + c
    qbase = wid * QPW

    def chunk(ch, carry):
        rbase = (qbase + ch * QC) * KNN
        pltpu.sync_copy(idx_hbm.at[pl.ds(rbase, RC)], idx_v)
        pltpu.async_copy(tgt_hbm.at[idx_v], rows_v, sem).wait()

        def qloop(q, carry2):
            def cloop(cc, carry3):
                sl = pl.ds(cc * 16, 16)
                v = ((rows_v[q * KNN + 0, sl] + rows_v[q * KNN + 1, sl])
                     + (rows_v[q * KNN + 2, sl] + rows_v[q * KNN + 3, sl]))
                acc_v[q, sl] = v * 0.25
                return carry3
            return lax.fori_loop(0, D // 16, cloop, carry2)
        lax.fori_loop(0, QC, qloop, 0)
        pltpu.sync_copy(acc_v, out_hbm.at[pl.ds(qbase + ch * QC, QC)])
        return carry
    lax.fori_loop(0, NCH, chunk, 0)


def _gather_mean(target_feats, idx_flat):
    mesh = plsc.VectorSubcoreMesh(core_axis_name="c", subcore_axis_name="s")
    f = functools.partial(
        pl.kernel,
        mesh=mesh,
        out_type=jax.ShapeDtypeStruct((Q, D), jnp.float32),
        scratch_types=[
            pltpu.VMEM((RC,), jnp.int32),
            pltpu.VMEM((RC, D), jnp.float32),
            pltpu.VMEM((QC, D), jnp.float32),
            pltpu.SemaphoreType.DMA,
        ],
    )(_gather_mean_body)
    return f(target_feats, idx_flat)


def kernel(source_feats, target_feats):
    eps = 1e-12
    src_norm = source_feats / jnp.maximum(
        jnp.linalg.norm(source_feats, axis=1, keepdims=True), eps)
    idx = _topk_indices(src_norm.astype(jnp.bfloat16), target_feats)
    return _gather_mean(target_feats, idx.reshape(Q * KNN))


# speedup vs baseline: 3.3348x; 1.0143x over previous
import functools

import jax
import jax.numpy as jnp
from jax import lax
from jax.experimental import pallas as pl
from jax.experimental.pallas import tpu as pltpu
from jax.experimental.pallas import tpu_sc as plsc

Q = 1024
D = 1024
K = 32768
KNN = 4
KB = 1024
NKB = K // KB
NEG = float("-inf")
BIGI = 2**30


BIGF = float(2**24)
RW = 128


def _topk_body(src_ref, tgt_ref, idx_out_ref, rv_ref, ri_ref, srcb_ref):
    kb = pl.program_id(0)

    @pl.when(kb == 0)
    def _init():
        rv_ref[...] = jnp.full((Q, RW), NEG, jnp.float32)
        ri_ref[...] = jnp.full((Q, RW), BIGF, jnp.float32)
        q = src_ref[...]
        qn = jnp.maximum(jnp.sqrt(jnp.sum(q * q, axis=1, keepdims=True)),
                         1e-12)
        srcb_ref[...] = (q / qn).astype(jnp.bfloat16)

    t = tgt_ref[...]
    nrm = jnp.maximum(jnp.sqrt(jnp.sum(t * t, axis=1, keepdims=True)), 1e-12)
    tb = (t / nrm).astype(jnp.bfloat16)
    s = lax.dot_general(
        srcb_ref[...], tb, (((1,), (1,)), ((), ())),
        preferred_element_type=jnp.float32)

    colf = (lax.broadcasted_iota(jnp.int32, (Q, KB), 1).astype(jnp.float32)
            + jnp.float32(kb * KB))

    x = s
    rv = rv_ref[...]
    ri = ri_ref[...]
    ms, as_ = [], []
    for _ in range(KNN):
        m = jnp.maximum(jnp.max(x, axis=1, keepdims=True),
                        jnp.max(rv, axis=1, keepdims=True))
        a = jnp.minimum(
            jnp.min(jnp.where(x == m, colf, BIGF), axis=1, keepdims=True),
            jnp.min(jnp.where(rv == m, ri, BIGF), axis=1, keepdims=True))
        ms.append(m)
        as_.append(a)
        x = jnp.where(colf == a, NEG, x)
        rv = jnp.where(ri == a, NEG, rv)

    lane = lax.broadcasted_iota(jnp.int32, (Q, RW), 1)
    nrv = jnp.full((Q, RW), NEG, jnp.float32)
    nri = jnp.full((Q, RW), BIGF, jnp.float32)
    for t_ in range(KNN):
        nrv = jnp.where(lane == t_, ms[t_], nrv)
        nri = jnp.where(lane == t_, as_[t_], nri)
    rv_ref[...] = nrv
    ri_ref[...] = nri

    @pl.when(kb == NKB - 1)
    def _out():
        idx_out_ref[...] = nri[:, :KNN].astype(jnp.int32)


def _topk_indices(src_f32, tgt_f32):
    return pl.pallas_call(
        _topk_body,
        grid=(NKB,),
        in_specs=[
            pl.BlockSpec((Q, D), lambda k: (0, 0)),
            pl.BlockSpec((KB, D), lambda k: (k, 0)),
        ],
        out_specs=pl.BlockSpec((Q, KNN), lambda k: (0, 0)),
        out_shape=jax.ShapeDtypeStruct((Q, KNN), jnp.int32),
        scratch_shapes=[
            pltpu.VMEM((Q, RW), jnp.float32),
            pltpu.VMEM((Q, RW), jnp.float32),
            pltpu.VMEM((Q, D), jnp.bfloat16),
        ],
    )(src_f32, tgt_f32)


NW = 32
QPW = Q // NW
QC = 8
NCH = QPW // QC
RC = QC * KNN


def _gather_mean_body(tgt_hbm, idx_hbm, out_hbm, idx_v, rows_v, acc_v, sem):
    c = lax.axis_index("c")
    s = lax.axis_index("s")
    wid = s * 2 + c
    qbase = wid * QPW

    def chunk(ch, carry):
        rbase = (qbase + ch * QC) * KNN
        pltpu.sync_copy(idx_hbm.at[pl.ds(rbase, RC)], idx_v)
        pltpu.async_copy(tgt_hbm.at[idx_v], rows_v, sem).wait()

        def qloop(q, carry2):
            def cloop(cc, carry3):
                sl = pl.ds(cc * 16, 16)
                v = ((rows_v[q * KNN + 0, sl] + rows_v[q * KNN + 1, sl])
                     + (rows_v[q * KNN + 2, sl] + rows_v[q * KNN + 3, sl]))
                acc_v[q, sl] = v * 0.25
                return carry3
            return lax.fori_loop(0, D // 16, cloop, carry2)
        lax.fori_loop(0, QC, qloop, 0)
        pltpu.sync_copy(acc_v, out_hbm.at[pl.ds(qbase + ch * QC, QC)])
        return carry
    lax.fori_loop(0, NCH, chunk, 0)


def _gather_mean(target_feats, idx_flat):
    mesh = plsc.VectorSubcoreMesh(core_axis_name="c", subcore_axis_name="s")
    f = functools.partial(
        pl.kernel,
        mesh=mesh,
        out_type=jax.ShapeDtypeStruct((Q, D), jnp.float32),
        scratch_types=[
            pltpu.VMEM((RC,), jnp.int32),
            pltpu.VMEM((RC, D), jnp.float32),
            pltpu.VMEM((QC, D), jnp.float32),
            pltpu.SemaphoreType.DMA,
        ],
    )(_gather_mean_body)
    return f(target_feats, idx_flat)


def kernel(source_feats, target_feats):
    idx = _topk_indices(source_feats, target_feats)
    return _gather_mean(target_feats, idx.reshape(Q * KNN))


# speedup vs baseline: 3.4491x; 1.0343x over previous
import functools

import jax
import jax.numpy as jnp
from jax import lax
from jax.experimental import pallas as pl
from jax.experimental.pallas import tpu as pltpu
from jax.experimental.pallas import tpu_sc as plsc

Q = 1024
D = 1024
K = 32768
KNN = 4
KB = 1024
NKB = K // KB
NEG = float("-inf")
BIGI = 2**30


BIGF = float(2**24)
RW = 128


def _topk_body(src_ref, tgt_ref, idx_out_ref, rv_ref, ri_ref, srcb_ref,
               col_ref):
    kb = pl.program_id(0)

    @pl.when(kb == 0)
    def _init():
        rv_ref[...] = jnp.full((Q, RW), NEG, jnp.float32)
        ri_ref[...] = jnp.full((Q, RW), BIGF, jnp.float32)
        q = src_ref[...]
        qn = jnp.maximum(jnp.sqrt(jnp.sum(q * q, axis=1, keepdims=True)),
                         1e-12)
        srcb_ref[...] = (q / qn).astype(jnp.bfloat16)
        col_ref[...] = lax.broadcasted_iota(
            jnp.int32, (Q, KB), 1).astype(jnp.float32)

    t = tgt_ref[...]
    nrm = jnp.maximum(jnp.sqrt(jnp.sum(t * t, axis=1, keepdims=True)), 1e-12)
    tb = (t / nrm).astype(jnp.bfloat16)
    s = lax.dot_general(
        srcb_ref[...], tb, (((1,), (1,)), ((), ())),
        preferred_element_type=jnp.float32)

    colf = col_ref[...]
    off = jnp.float32(kb * KB)

    x = s
    rv = rv_ref[...]
    ri = ri_ref[...]
    ms, as_ = [], []
    for t_ in range(KNN):
        m = jnp.maximum(jnp.max(x, axis=1, keepdims=True),
                        jnp.max(rv, axis=1, keepdims=True))
        a = jnp.minimum(
            jnp.min(jnp.where(x == m, colf, BIGF), axis=1, keepdims=True)
            + off,
            jnp.min(jnp.where(rv == m, ri, BIGF), axis=1, keepdims=True))
        ms.append(m)
        as_.append(a)
        if t_ < KNN - 1:
            x = jnp.where(colf == a - off, NEG, x)
            rv = jnp.where(ri == a, NEG, rv)

    lane = lax.broadcasted_iota(jnp.int32, (Q, RW), 1)
    nrv = jnp.full((Q, RW), NEG, jnp.float32)
    nri = jnp.full((Q, RW), BIGF, jnp.float32)
    for t_ in range(KNN):
        nrv = jnp.where(lane == t_, ms[t_], nrv)
        nri = jnp.where(lane == t_, as_[t_], nri)
    rv_ref[...] = nrv
    ri_ref[...] = nri

    @pl.when(kb == NKB - 1)
    def _out():
        idx_out_ref[...] = nri[:, :KNN].astype(jnp.int32)


def _topk_indices(src_f32, tgt_f32):
    return pl.pallas_call(
        _topk_body,
        grid=(NKB,),
        in_specs=[
            pl.BlockSpec((Q, D), lambda k: (0, 0)),
            pl.BlockSpec((KB, D), lambda k: (k, 0)),
        ],
        out_specs=pl.BlockSpec((Q, KNN), lambda k: (0, 0)),
        out_shape=jax.ShapeDtypeStruct((Q, KNN), jnp.int32),
        scratch_shapes=[
            pltpu.VMEM((Q, RW), jnp.float32),
            pltpu.VMEM((Q, RW), jnp.float32),
            pltpu.VMEM((Q, D), jnp.bfloat16),
            pltpu.VMEM((Q, KB), jnp.float32),
        ],
    )(src_f32, tgt_f32)


NW = 32
QPW = Q // NW
QC = 8
NCH = QPW // QC
RC = QC * KNN


def _gather_mean_body(tgt_hbm, idx_hbm, out_hbm,
                      idx0, idx1, rows0, rows1, acc_v, sem0, sem1):
    c = lax.axis_index("c")
    s = lax.axis_index("s")
    wid = s * 2 + c
    qbase = wid * QPW
    idx_b = (idx0, idx1)
    rows_b = (rows0, rows1)
    sem_b = (sem0, sem1)

    def start(ch):
        b = ch % 2
        rbase = (qbase + ch * QC) * KNN
        pltpu.sync_copy(idx_hbm.at[pl.ds(rbase, RC)], idx_b[b])
        pltpu.async_copy(tgt_hbm.at[idx_b[b]], rows_b[b], sem_b[b])

    start(0)
    for ch in range(NCH):
        b = ch % 2
        rows_v = rows_b[b]
        pltpu.make_async_copy(tgt_hbm.at[idx_b[b]], rows_v, sem_b[b]).wait()
        if ch + 1 < NCH:
            start(ch + 1)

        def cloop(cc, carry):
            sl = pl.ds(cc * 16, 16)
            for q in range(QC):
                v = ((rows_v[q * KNN + 0, sl] + rows_v[q * KNN + 1, sl])
                     + (rows_v[q * KNN + 2, sl] + rows_v[q * KNN + 3, sl]))
                acc_v[q, sl] = v * 0.25
            return carry
        lax.fori_loop(0, D // 16, cloop, 0)
        pltpu.sync_copy(acc_v, out_hbm.at[pl.ds(qbase + ch * QC, QC)])


def _gather_mean(target_feats, idx_flat):
    mesh = plsc.VectorSubcoreMesh(core_axis_name="c", subcore_axis_name="s")
    f = functools.partial(
        pl.kernel,
        mesh=mesh,
        out_type=jax.ShapeDtypeStruct((Q, D), jnp.float32),
        scratch_types=[
            pltpu.VMEM((RC,), jnp.int32),
            pltpu.VMEM((RC,), jnp.int32),
            pltpu.VMEM((RC, D), jnp.float32),
            pltpu.VMEM((RC, D), jnp.float32),
            pltpu.VMEM((QC, D), jnp.float32),
            pltpu.SemaphoreType.DMA,
            pltpu.SemaphoreType.DMA,
        ],
    )(_gather_mean_body)
    return f(target_feats, idx_flat)


def kernel(source_feats, target_feats):
    idx = _topk_indices(source_feats, target_feats)
    return _gather_mean(target_feats, idx.reshape(Q * KNN))


# speedup vs baseline: 3.6863x; 1.0688x over previous
import functools

import jax
import jax.numpy as jnp
from jax import lax
from jax.experimental import pallas as pl
from jax.experimental.pallas import tpu as pltpu
from jax.experimental.pallas import tpu_sc as plsc

Q = 1024
D = 1024
K = 32768
KNN = 4
KB = 2048
NKB = K // KB
NEG = float("-inf")
BIGI = 2**30


BIGF = float(2**24)
RW = 128


def _topk_body(src_ref, tgt_ref, idx_out_ref, rv_ref, ri_ref, srcb_ref,
               col_ref):
    kb = pl.program_id(0)

    @pl.when(kb == 0)
    def _init():
        rv_ref[...] = jnp.full((Q, RW), NEG, jnp.float32)
        ri_ref[...] = jnp.full((Q, RW), BIGF, jnp.float32)
        q = src_ref[...]
        qn = jnp.maximum(jnp.sqrt(jnp.sum(q * q, axis=1, keepdims=True)),
                         1e-12)
        srcb_ref[...] = (q / qn).astype(jnp.bfloat16)
        col_ref[...] = lax.broadcasted_iota(
            jnp.int32, (Q, KB), 1).astype(jnp.float32)

    t = tgt_ref[...]
    nrm = jnp.maximum(jnp.sqrt(jnp.sum(t * t, axis=1, keepdims=True)), 1e-12)
    tb = (t / nrm).astype(jnp.bfloat16)
    s = lax.dot_general(
        srcb_ref[...], tb, (((1,), (1,)), ((), ())),
        preferred_element_type=jnp.float32)

    colf = col_ref[...]
    off = jnp.float32(kb * KB)

    x = s
    rv = rv_ref[...]
    ri = ri_ref[...]
    ms, as_ = [], []
    for t_ in range(KNN):
        m = jnp.maximum(jnp.max(x, axis=1, keepdims=True),
                        jnp.max(rv, axis=1, keepdims=True))
        a = jnp.minimum(
            jnp.min(jnp.where(x == m, colf, BIGF), axis=1, keepdims=True)
            + off,
            jnp.min(jnp.where(rv == m, ri, BIGF), axis=1, keepdims=True))
        ms.append(m)
        as_.append(a)
        if t_ < KNN - 1:
            x = jnp.where(colf == a - off, NEG, x)
            rv = jnp.where(ri == a, NEG, rv)

    lane = lax.broadcasted_iota(jnp.int32, (Q, RW), 1)
    nrv = jnp.full((Q, RW), NEG, jnp.float32)
    nri = jnp.full((Q, RW), BIGF, jnp.float32)
    for t_ in range(KNN):
        nrv = jnp.where(lane == t_, ms[t_], nrv)
        nri = jnp.where(lane == t_, as_[t_], nri)
    rv_ref[...] = nrv
    ri_ref[...] = nri

    @pl.when(kb == NKB - 1)
    def _out():
        idx_out_ref[...] = nri[:, :KNN].astype(jnp.int32)


def _topk_indices(src_f32, tgt_f32):
    return pl.pallas_call(
        _topk_body,
        grid=(NKB,),
        in_specs=[
            pl.BlockSpec((Q, D), lambda k: (0, 0)),
            pl.BlockSpec((KB, D), lambda k: (k, 0)),
        ],
        out_specs=pl.BlockSpec((Q, KNN), lambda k: (0, 0)),
        out_shape=jax.ShapeDtypeStruct((Q, KNN), jnp.int32),
        scratch_shapes=[
            pltpu.VMEM((Q, RW), jnp.float32),
            pltpu.VMEM((Q, RW), jnp.float32),
            pltpu.VMEM((Q, D), jnp.bfloat16),
            pltpu.VMEM((Q, KB), jnp.float32),
        ],
    )(src_f32, tgt_f32)


NW = 32
QPW = Q // NW
QC = 8
NCH = QPW // QC
RC = QC * KNN


def _gather_mean_body(tgt_hbm, idx_hbm, out_hbm,
                      idx0, idx1, rows0, rows1, acc_v, sem0, sem1):
    c = lax.axis_index("c")
    s = lax.axis_index("s")
    wid = s * 2 + c
    qbase = wid * QPW
    idx_b = (idx0, idx1)
    rows_b = (rows0, rows1)
    sem_b = (sem0, sem1)

    def start(ch):
        b = ch % 2
        rbase = (qbase + ch * QC) * KNN
        pltpu.sync_copy(idx_hbm.at[pl.ds(rbase, RC)], idx_b[b])
        pltpu.async_copy(tgt_hbm.at[idx_b[b]], rows_b[b], sem_b[b])

    start(0)
    for ch in range(NCH):
        b = ch % 2
        rows_v = rows_b[b]
        pltpu.make_async_copy(tgt_hbm.at[idx_b[b]], rows_v, sem_b[b]).wait()
        if ch + 1 < NCH:
            start(ch + 1)

        def cloop(cc, carry):
            sl = pl.ds(cc * 16, 16)
            for q in range(QC):
                v = ((rows_v[q * KNN + 0, sl] + rows_v[q * KNN + 1, sl])
                     + (rows_v[q * KNN + 2, sl] + rows_v[q * KNN + 3, sl]))
                acc_v[q, sl] = v * 0.25
            return carry
        lax.fori_loop(0, D // 16, cloop, 0)
        pltpu.sync_copy(acc_v, out_hbm.at[pl.ds(qbase + ch * QC, QC)])


def _gather_mean(target_feats, idx_flat):
    mesh = plsc.VectorSubcoreMesh(core_axis_name="c", subcore_axis_name="s")
    f = functools.partial(
        pl.kernel,
        mesh=mesh,
        out_type=jax.ShapeDtypeStruct((Q, D), jnp.float32),
        scratch_types=[
            pltpu.VMEM((RC,), jnp.int32),
            pltpu.VMEM((RC,), jnp.int32),
            pltpu.VMEM((RC, D), jnp.float32),
            pltpu.VMEM((RC, D), jnp.float32),
            pltpu.VMEM((QC, D), jnp.float32),
            pltpu.SemaphoreType.DMA,
            pltpu.SemaphoreType.DMA,
        ],
    )(_gather_mean_body)
    return f(target_feats, idx_flat)


def kernel(source_feats, target_feats):
    idx = _topk_indices(source_feats, target_feats)
    return _gather_mean(target_feats, idx.reshape(Q * KNN))


# speedup vs baseline: 3.7678x; 1.0221x over previous
import functools

import jax
import jax.numpy as jnp
from jax import lax
from jax.experimental import pallas as pl
from jax.experimental.pallas import tpu as pltpu
from jax.experimental.pallas import tpu_sc as plsc

Q = 1024
D = 1024
K = 32768
KNN = 4
KB = 2048
NKB = K // KB
NEG = float("-inf")
BIGI = 2**30


BIGF = float(2**24)
RW = 128


def _topk_body(src_ref, tgt_ref, idx_out_ref, rv_ref, ri_ref, srcb_ref):
    kb = pl.program_id(0)

    @pl.when(kb == 0)
    def _init():
        rv_ref[...] = jnp.full((Q, RW), NEG, jnp.float32)
        ri_ref[...] = jnp.full((Q, RW), BIGF, jnp.float32)
        q = src_ref[...]
        qn = jnp.maximum(jnp.sqrt(jnp.sum(q * q, axis=1, keepdims=True)),
                         1e-12)
        srcb_ref[...] = (q / qn).astype(jnp.bfloat16)

    t = tgt_ref[...]
    nrm = jnp.maximum(jnp.sqrt(jnp.sum(t * t, axis=1, keepdims=True)), 1e-12)
    tb = (t / nrm).astype(jnp.bfloat16)
    s = lax.dot_general(
        srcb_ref[...], tb, (((1,), (1,)), ((), ())),
        preferred_element_type=jnp.float32)

    colf = lax.broadcasted_iota(jnp.int32, (Q, KB), 1).astype(jnp.float32)
    off = jnp.float32(kb * KB)

    x = s
    rv = rv_ref[...]
    ri = ri_ref[...]
    ms, as_ = [], []
    for t_ in range(KNN):
        m = jnp.maximum(jnp.max(x, axis=1, keepdims=True),
                        jnp.max(rv, axis=1, keepdims=True))
        a = jnp.minimum(
            jnp.min(jnp.where(x == m, colf, BIGF), axis=1, keepdims=True)
            + off,
            jnp.min(jnp.where(rv == m, ri, BIGF), axis=1, keepdims=True))
        ms.append(m)
        as_.append(a)
        if t_ < KNN - 1:
            x = jnp.where(colf == a - off, NEG, x)
            rv = jnp.where(ri == a, NEG, rv)

    lane = lax.broadcasted_iota(jnp.int32, (Q, RW), 1)
    nrv = jnp.full((Q, RW), NEG, jnp.float32)
    nri = jnp.full((Q, RW), BIGF, jnp.float32)
    for t_ in range(KNN):
        nrv = jnp.where(lane == t_, ms[t_], nrv)
        nri = jnp.where(lane == t_, as_[t_], nri)
    rv_ref[...] = nrv
    ri_ref[...] = nri

    @pl.when(kb == NKB - 1)
    def _out():
        idx_out_ref[...] = nri[:, :KNN].astype(jnp.int32)


def _topk_indices(src_f32, tgt_f32):
    return pl.pallas_call(
        _topk_body,
        grid=(NKB,),
        in_specs=[
            pl.BlockSpec((Q, D), lambda k: (0, 0)),
            pl.BlockSpec((KB, D), lambda k: (k, 0)),
        ],
        out_specs=pl.BlockSpec((Q, KNN), lambda k: (0, 0)),
        out_shape=jax.ShapeDtypeStruct((Q, KNN), jnp.int32),
        scratch_shapes=[
            pltpu.VMEM((Q, RW), jnp.float32),
            pltpu.VMEM((Q, RW), jnp.float32),
            pltpu.VMEM((Q, D), jnp.bfloat16),
        ],
    )(src_f32, tgt_f32)


NW = 32
QPW = Q // NW
QC = 8
NCH = QPW // QC
RC = QC * KNN


def _gather_mean_body(tgt_hbm, idx_hbm, out_hbm,
                      idx0, idx1, rows0, rows1, acc_v, sem0, sem1):
    c = lax.axis_index("c")
    s = lax.axis_index("s")
    wid = s * 2 + c
    qbase = wid * QPW
    idx_b = (idx0, idx1)
    rows_b = (rows0, rows1)
    sem_b = (sem0, sem1)

    def start(ch):
        b = ch % 2
        rbase = (qbase + ch * QC) * KNN
        pltpu.sync_copy(idx_hbm.at[pl.ds(rbase, RC)], idx_b[b])
        pltpu.async_copy(tgt_hbm.at[idx_b[b]], rows_b[b], sem_b[b])

    start(0)
    for ch in range(NCH):
        b = ch % 2
        rows_v = rows_b[b]
        pltpu.make_async_copy(tgt_hbm.at[idx_b[b]], rows_v, sem_b[b]).wait()
        if ch + 1 < NCH:
            start(ch + 1)

        def cloop(cc, carry):
            sl = pl.ds(cc * 16, 16)
            for q in range(QC):
                v = ((rows_v[q * KNN + 0, sl] + rows_v[q * KNN + 1, sl])
                     + (rows_v[q * KNN + 2, sl] + rows_v[q * KNN + 3, sl]))
                acc_v[q, sl] = v * 0.25
            return carry
        lax.fori_loop(0, D // 16, cloop, 0)
        pltpu.sync_copy(acc_v, out_hbm.at[pl.ds(qbase + ch * QC, QC)])


def _gather_mean(target_feats, idx_flat):
    mesh = plsc.VectorSubcoreMesh(core_axis_name="c", subcore_axis_name="s")
    f = functools.partial(
        pl.kernel,
        mesh=mesh,
        out_type=jax.ShapeDtypeStruct((Q, D), jnp.float32),
        scratch_types=[
            pltpu.VMEM((RC,), jnp.int32),
            pltpu.VMEM((RC,), jnp.int32),
            pltpu.VMEM((RC, D), jnp.float32),
            pltpu.VMEM((RC, D), jnp.float32),
            pltpu.VMEM((QC, D), jnp.float32),
            pltpu.SemaphoreType.DMA,
            pltpu.SemaphoreType.DMA,
        ],
    )(_gather_mean_body)
    return f(target_feats, idx_flat)


def kernel(source_feats, target_feats):
    idx = _topk_indices(source_feats, target_feats)
    return _gather_mean(target_feats, idx.reshape(Q * KNN))


# speedup vs baseline: 3.8293x; 1.0163x over previous
import functools

import jax
import jax.numpy as jnp
from jax import lax
from jax.experimental import pallas as pl
from jax.experimental.pallas import tpu as pltpu
from jax.experimental.pallas import tpu_sc as plsc

Q = 1024
D = 1024
K = 32768
KNN = 4
KB = 4096
NKB = K // KB
NEG = float("-inf")
BIGI = 2**30


BIGF = float(2**24)
RW = 128


def _topk_body(src_ref, tgt_ref, idx_out_ref, rv_ref, ri_ref, srcb_ref):
    kb = pl.program_id(0)

    @pl.when(kb == 0)
    def _init():
        rv_ref[...] = jnp.full((Q, RW), NEG, jnp.float32)
        ri_ref[...] = jnp.full((Q, RW), BIGF, jnp.float32)
        q = src_ref[...]
        qn = jnp.maximum(jnp.sqrt(jnp.sum(q * q, axis=1, keepdims=True)),
                         1e-12)
        srcb_ref[...] = (q / qn).astype(jnp.bfloat16)

    t = tgt_ref[...]
    nrm = jnp.maximum(jnp.sqrt(jnp.sum(t * t, axis=1, keepdims=True)), 1e-12)
    tb = (t / nrm).astype(jnp.bfloat16)
    s = lax.dot_general(
        srcb_ref[...], tb, (((1,), (1,)), ((), ())),
        preferred_element_type=jnp.float32)

    colf = lax.broadcasted_iota(jnp.int32, (Q, KB), 1).astype(jnp.float32)
    off = jnp.float32(kb * KB)

    x = s
    rv = rv_ref[...]
    ri = ri_ref[...]
    ms, as_ = [], []
    for t_ in range(KNN):
        m = jnp.maximum(jnp.max(x, axis=1, keepdims=True),
                        jnp.max(rv, axis=1, keepdims=True))
        a = jnp.minimum(
            jnp.min(jnp.where(x == m, colf, BIGF), axis=1, keepdims=True)
            + off,
            jnp.min(jnp.where(rv == m, ri, BIGF), axis=1, keepdims=True))
        ms.append(m)
        as_.append(a)
        if t_ < KNN - 1:
            x = jnp.where(colf == a - off, NEG, x)
            rv = jnp.where(ri == a, NEG, rv)

    lane = lax.broadcasted_iota(jnp.int32, (Q, RW), 1)
    nrv = jnp.full((Q, RW), NEG, jnp.float32)
    nri = jnp.full((Q, RW), BIGF, jnp.float32)
    for t_ in range(KNN):
        nrv = jnp.where(lane == t_, ms[t_], nrv)
        nri = jnp.where(lane == t_, as_[t_], nri)
    rv_ref[...] = nrv
    ri_ref[...] = nri

    @pl.when(kb == NKB - 1)
    def _out():
        idx_out_ref[...] = nri[:, :KNN].astype(jnp.int32)


def _topk_indices(src_f32, tgt_f32):
    return pl.pallas_call(
        _topk_body,
        grid=(NKB,),
        in_specs=[
            pl.BlockSpec((Q, D), lambda k: (0, 0)),
            pl.BlockSpec((KB, D), lambda k: (k, 0)),
        ],
        out_specs=pl.BlockSpec((Q, KNN), lambda k: (0, 0)),
        out_shape=jax.ShapeDtypeStruct((Q, KNN), jnp.int32),
        scratch_shapes=[
            pltpu.VMEM((Q, RW), jnp.float32),
            pltpu.VMEM((Q, RW), jnp.float32),
            pltpu.VMEM((Q, D), jnp.bfloat16),
        ],
        compiler_params=pltpu.CompilerParams(
            vmem_limit_bytes=110 * 1024 * 1024),
    )(src_f32, tgt_f32)


NW = 32
QPW = Q // NW
QC = 8
NCH = QPW // QC
RC = QC * KNN


def _gather_mean_body(tgt_hbm, idx_hbm, out_hbm,
                      idx0, idx1, rows0, rows1, acc_v, sem0, sem1):
    c = lax.axis_index("c")
    s = lax.axis_index("s")
    wid = s * 2 + c
    qbase = wid * QPW
    idx_b = (idx0, idx1)
    rows_b = (rows0, rows1)
    sem_b = (sem0, sem1)

    def start(ch):
        b = ch % 2
        rbase = (qbase + ch * QC) * KNN
        pltpu.sync_copy(idx_hbm.at[pl.ds(rbase, RC)], idx_b[b])
        pltpu.async_copy(tgt_hbm.at[idx_b[b]], rows_b[b], sem_b[b])

    start(0)
    for ch in range(NCH):
        b = ch % 2
        rows_v = rows_b[b]
        pltpu.make_async_copy(tgt_hbm.at[idx_b[b]], rows_v, sem_b[b]).wait()
        if ch + 1 < NCH:
            start(ch + 1)

        def cloop(cc, carry):
            sl = pl.ds(cc * 16, 16)
            for q in range(QC):
                v = ((rows_v[q * KNN + 0, sl] + rows_v[q * KNN + 1, sl])
                     + (rows_v[q * KNN + 2, sl] + rows_v[q * KNN + 3, sl]))
                acc_v[q, sl] = v * 0.25
            return carry
        lax.fori_loop(0, D // 16, cloop, 0)
        pltpu.sync_copy(acc_v, out_hbm.at[pl.ds(qbase + ch * QC, QC)])


def _gather_mean(target_feats, idx_flat):
    mesh = plsc.VectorSubcoreMesh(core_axis_name="c", subcore_axis_name="s")
    f = functools.partial(
        pl.kernel,
        mesh=mesh,
        out_type=jax.ShapeDtypeStruct((Q, D), jnp.float32),
        scratch_types=[
            pltpu.VMEM((RC,), jnp.int32),
            pltpu.VMEM((RC,), jnp.int32),
            pltpu.VMEM((RC, D), jnp.float32),
            pltpu.VMEM((RC, D), jnp.float32),
            pltpu.VMEM((QC, D), jnp.float32),
            pltpu.SemaphoreType.DMA,
            pltpu.SemaphoreType.DMA,
        ],
    )(_gather_mean_body)
    return f(target_feats, idx_flat)


def kernel(source_feats, target_feats):
    idx = _topk_indices(source_feats, target_feats)
    return _gather_mean(target_feats, idx.reshape(Q * KNN))
